# baseline (speedup 1.0000x reference)
import jax
import jax.numpy as jnp
from jax import lax
from jax.experimental import pallas as pl
from jax.experimental.pallas import tpu as pltpu
from jax.experimental.pallas import tpu_sc as plsc

NC = 2
NS = 16
L = 16
NW = NC * NS
B = 16384
D = 32
BPW = B // NW
CH = 128
NCH = BPW // CH
NUM_R16 = 1000000 // 16


def _sc_body(iso_hbm, rand_hbm, idx1_hbm, idx2_hbm, ridx1_hbm, ridx2_hbm,
             beta_hbm, out_hbm,
             idx1_v, idx2_v, ridx1_v, ridx2_v, z1_v, z2_v, r1_v, r2_v,
             beta_v, out_v, sem):
    wid = lax.axis_index("s") * NC + lax.axis_index("c")

    pltpu.sync_copy(idx1_hbm.at[wid], idx1_v)
    pltpu.sync_copy(idx2_hbm.at[wid], idx2_v)
    pltpu.sync_copy(ridx1_hbm.at[wid], ridx1_v)
    pltpu.sync_copy(ridx2_hbm.at[wid], ridx2_v)
    pltpu.sync_copy(beta_hbm, beta_v)

    copies = []
    for j in range(NCH):
        sl = pl.ds(j * CH, CH)
        copies.append(pltpu.async_copy(iso_hbm.at[idx1_v.at[j]], z1_v.at[sl], sem))
        copies.append(pltpu.async_copy(iso_hbm.at[idx2_v.at[j]], z2_v.at[sl], sem))
        copies.append(pltpu.async_copy(rand_hbm.at[ridx1_v.at[j]], r1_v.at[sl], sem))
        copies.append(pltpu.async_copy(rand_hbm.at[ridx2_v.at[j]], r2_v.at[sl], sem))
    for c in copies:
        c.wait()

    beta_vec = beta_v[...]
    iota = lax.iota(jnp.int32, L)
    mask15 = jnp.full((L,), 15, jnp.int32)

    def group(g, carry):
        rows = g * L + iota
        chunk = rows >> 7
        lane = rows & jnp.full((L,), 127, jnp.int32)
        p1v = plsc.load_gather(idx1_v, [chunk, lane])
        p2v = plsc.load_gather(idx2_v, [chunk, lane])
        acc = jnp.zeros((L,), jnp.float32)
        for d in range(D):
            col = jnp.full((L,), d, jnp.int32)
            a = plsc.load_gather(z1_v, [rows, col])
            b = plsc.load_gather(z2_v, [rows, col])
            df = a - b
            acc = acc + df * df
        r1 = plsc.load_gather(r1_v, [rows, p1v & mask15])
        r2 = plsc.load_gather(r2_v, [rows, p2v & mask15])
        seed = jnp.int32(0x5F3759DF) - (plsc.bitcast(acc, jnp.int32) >> 1)
        y = plsc.bitcast(seed, jnp.float32)
        h = acc * jnp.float32(0.5)
        for _ in range(3):
            y = y * (jnp.float32(1.5) - h * y * y)
        dist = acc * y
        out_v[pl.ds(g * L, L)] = r1 + r2 - beta_vec * dist
        return carry

    lax.fori_loop(0, BPW // L, group, 0)
    pltpu.sync_copy(out_v, out_hbm.at[pl.ds(wid * BPW, BPW)])


def kernel(protein1_idx, protein2_idx, iso_emb, rand_eff, beta_iso):
    p1 = protein1_idx.astype(jnp.int32)
    p2 = protein2_idx.astype(jnp.int32)
    idx1 = p1.reshape(NW, NCH, CH)
    idx2 = p2.reshape(NW, NCH, CH)
    ridx1 = (p1 >> 4).reshape(NW, NCH, CH)
    ridx2 = (p2 >> 4).reshape(NW, NCH, CH)
    rand16 = rand_eff.astype(jnp.float32).reshape(NUM_R16, 16)
    beta = jnp.full((L,), beta_iso, jnp.float32)
    mesh = plsc.VectorSubcoreMesh(
        core_axis_name="c", subcore_axis_name="s",
        num_cores=NC, num_subcores=NS)
    run = pl.kernel(
        _sc_body,
        out_type=jax.ShapeDtypeStruct((B,), jnp.float32),
        mesh=mesh,
        compiler_params=pltpu.CompilerParams(
            needs_layout_passes=False, use_tc_tiling_on_sc=False),
        scratch_types=[
            pltpu.VMEM((NCH, CH), jnp.int32),
            pltpu.VMEM((NCH, CH), jnp.int32),
            pltpu.VMEM((NCH, CH), jnp.int32),
            pltpu.VMEM((NCH, CH), jnp.int32),
            pltpu.VMEM((BPW, D), jnp.float32),
            pltpu.VMEM((BPW, D), jnp.float32),
            pltpu.VMEM((BPW, 16), jnp.float32),
            pltpu.VMEM((BPW, 16), jnp.float32),
            pltpu.VMEM((L,), jnp.float32),
            pltpu.VMEM((BPW,), jnp.float32),
            pltpu.SemaphoreType.DMA,
        ],
    )
    return run(iso_emb.astype(jnp.float32), rand16, idx1, idx2,
               ridx1, ridx2, beta)
